# baseline (speedup 1.0000x reference)
import jax
import jax.numpy as jnp
from jax import lax
from jax.experimental import pallas as pl
from jax.experimental.pallas import tpu as pltpu

B = 128
N = 100000
W = 2048
K = 4
NB = (N + W - 1) // W
_REM = NB - K
_GEND = 1 + (_REM + K - 1) // K
_GRID = _GEND + 1
_LAST_VALID = (NB - 2) - (3 + K * (_GEND - 2)) + 1


def _topk_kernel(t_sm, *refs):
    x_refs = refs[:K]
    xany_ref, t_ref, out1_ref, out5_ref, acc_ref, v_ref, vbuf_ref, sem = refs[K:]
    j = pl.program_id(0)

    @pl.when(j == 0)
    def _gather():
        copies = []
        for i in range(B):
            col0 = pl.multiple_of(
                jnp.minimum((t_sm[i] // 128) * 128, 128 * (N // 128) - 128),
                128)
            c = pltpu.make_async_copy(
                xany_ref.at[pl.ds(8 * (i // 8), 8), pl.ds(col0, 128)],
                vbuf_ref.at[i],
                sem,
            )
            c.start()
            copies.append(c)
        for c in copies:
            c.wait()
        t2 = t_ref[...]
        col0v = jnp.minimum((t2 // 128) * 128, 128 * (N // 128) - 128)
        lane = t2 - col0v
        rmod = lax.broadcasted_iota(jnp.int32, (B, 8, 128), 0) % 8
        smask = lax.broadcasted_iota(jnp.int32, (B, 8, 128), 1) == rmod
        lane3 = lax.broadcast_in_dim(lane, (B, 8, 128), (0, 1))
        lmask = lax.broadcasted_iota(jnp.int32, (B, 8, 128), 2) == lane3
        picked = jnp.where(smask & lmask, vbuf_ref[...], 0.0)
        v_ref[...] = jnp.sum(jnp.sum(picked, axis=2), axis=1, keepdims=True)
        acc_ref[...] = jnp.zeros_like(acc_ref)

    li = lax.broadcasted_iota(jnp.int32, (1, W), 1)
    ones = jnp.ones((W, 1), jnp.float32)

    def _hit(x, vv, tl, lim=None):
        eq = (x == vv) & (li < tl)
        gt = x > vv
        if lim is not None:
            gt = gt & (li < lim)
        return jnp.where(gt | eq, 1.0, 0.0)

    @pl.when(j == 1)
    def _first():
        t2 = t_ref[...]
        tl_tail = t2 - (NB - 1) * W
        pick = jnp.where(li == tl_tail, x_refs[0][...], 0.0)
        v_new = jnp.sum(pick, axis=1, keepdims=True)
        v_ref[...] = jnp.where(tl_tail >= 0, v_new, v_ref[...])
        vv = v_ref[...]
        h = _hit(x_refs[0][...], vv, tl_tail, lim=N - (NB - 1) * W)
        for k in range(1, K):
            h = h + _hit(x_refs[k][...], vv, t2 - (k - 1) * W)
        acc_ref[...] += lax.dot_general(
            h, ones, (((1,), (0,)), ((), ())),
            preferred_element_type=jnp.float32)

    @pl.when(jnp.logical_and(j >= 2, j < _GEND))
    def _middle():
        t2 = t_ref[...]
        vv = v_ref[...]
        h = None
        for k in range(K):
            tl = t2 - (3 + K * (j - 2) + k) * W
            hk = _hit(x_refs[k][...], vv, tl)
            h = hk if h is None else h + hk
        acc_ref[...] += lax.dot_general(
            h, ones, (((1,), (0,)), ((), ())),
            preferred_element_type=jnp.float32)

    @pl.when(j == _GEND)
    def _last():
        t2 = t_ref[...]
        vv = v_ref[...]
        h = None
        for k in range(_LAST_VALID):
            tl = t2 - (3 + K * (j - 2) + k) * W
            hk = _hit(x_refs[k][...], vv, tl)
            h = hk if h is None else h + hk
        rank = acc_ref[...] + lax.dot_general(
            h, ones, (((1,), (0,)), ((), ())),
            preferred_element_type=jnp.float32)
        out1_ref[0, 0] = jnp.sum(jnp.where(rank < 1.0, 1.0, 0.0)) * (100.0 / B)
        out5_ref[0, 0] = jnp.sum(jnp.where(rank < 5.0, 1.0, 0.0)) * (100.0 / B)


def _mk_imap(k):
    first = NB - 1 if k == 0 else k - 1

    def imap(j, ts):
        b = jnp.minimum(3 + K * (j - 2) + k, NB - 2)
        return (0, jnp.where(j <= 1, first, b))

    return imap


def _topk_acc(x, t2):
    grid_spec = pltpu.PrefetchScalarGridSpec(
        num_scalar_prefetch=1,
        grid=(_GRID,),
        in_specs=(
            [pl.BlockSpec((B, W), _mk_imap(k)) for k in range(K)]
            + [pl.BlockSpec(memory_space=pl.ANY),
               pl.BlockSpec((B, 1), lambda j, ts: (0, 0))]
        ),
        out_specs=[
            pl.BlockSpec(memory_space=pltpu.SMEM),
            pl.BlockSpec(memory_space=pltpu.SMEM),
        ],
        scratch_shapes=[
            pltpu.VMEM((B, 1), jnp.float32),
            pltpu.VMEM((B, 1), jnp.float32),
            pltpu.VMEM((B, 8, 128), jnp.float32),
            pltpu.SemaphoreType.DMA,
        ],
    )
    return pl.pallas_call(
        _topk_kernel,
        grid_spec=grid_spec,
        out_shape=[
            jax.ShapeDtypeStruct((1, 1), jnp.float32),
            jax.ShapeDtypeStruct((1, 1), jnp.float32),
        ],
        compiler_params=pltpu.CompilerParams(
            dimension_semantics=("arbitrary",)),
    )(t2.reshape(B), *([x] * K), x, t2)


def kernel(output, target):
    t32 = target.astype(jnp.int32)
    r1, r5 = _topk_acc(output, t32.reshape(B, 1))
    return (r1.reshape(1), r5.reshape(1))


# speedup vs baseline: 1.0422x; 1.0422x over previous
import jax
import jax.numpy as jnp
from jax import lax
from jax.experimental import pallas as pl
from jax.experimental.pallas import tpu as pltpu

B = 128
N = 100000
W = 4096
K = 4
NB = (N + W - 1) // W
_REM = NB - K
_GEND = 1 + (_REM + K - 1) // K
_GRID = _GEND + 1
_LAST_VALID = (NB - 2) - (3 + K * (_GEND - 2)) + 1


def _topk_kernel(t_sm, *refs):
    x_refs = refs[:K]
    xany_ref, t_ref, out1_ref, out5_ref, acc_ref, v_ref, vbuf_ref, sem = refs[K:]
    j = pl.program_id(0)

    @pl.when(j == 0)
    def _gather():
        copies = []
        for i in range(B):
            col0 = pl.multiple_of(
                jnp.minimum((t_sm[i] // 128) * 128, 128 * (N // 128) - 128),
                128)
            c = pltpu.make_async_copy(
                xany_ref.at[pl.ds(8 * (i // 8), 8), pl.ds(col0, 128)],
                vbuf_ref.at[i],
                sem,
            )
            c.start()
            copies.append(c)
        for c in copies:
            c.wait()
        t2 = t_ref[...]
        col0v = jnp.minimum((t2 // 128) * 128, 128 * (N // 128) - 128)
        lane = t2 - col0v
        rmod = lax.broadcasted_iota(jnp.int32, (B, 8, 128), 0) % 8
        smask = lax.broadcasted_iota(jnp.int32, (B, 8, 128), 1) == rmod
        lane3 = lax.broadcast_in_dim(lane, (B, 8, 128), (0, 1))
        lmask = lax.broadcasted_iota(jnp.int32, (B, 8, 128), 2) == lane3
        picked = jnp.where(smask & lmask, vbuf_ref[...], 0.0)
        v_ref[...] = jnp.sum(jnp.sum(picked, axis=2), axis=1, keepdims=True)
        acc_ref[...] = jnp.zeros_like(acc_ref)

    li = lax.broadcasted_iota(jnp.int32, (1, W), 1)
    ones = jnp.ones((W, 1), jnp.float32)

    def _hit(x, vv, tl, lim=None):
        eq = (x == vv) & (li < tl)
        gt = x > vv
        if lim is not None:
            gt = gt & (li < lim)
        return jnp.where(gt | eq, 1.0, 0.0)

    @pl.when(j == 1)
    def _first():
        t2 = t_ref[...]
        tl_tail = t2 - (NB - 1) * W
        pick = jnp.where(li == tl_tail, x_refs[0][...], 0.0)
        v_new = jnp.sum(pick, axis=1, keepdims=True)
        v_ref[...] = jnp.where(tl_tail >= 0, v_new, v_ref[...])
        vv = v_ref[...]
        h = _hit(x_refs[0][...], vv, tl_tail, lim=N - (NB - 1) * W)
        for k in range(1, K):
            h = h + _hit(x_refs[k][...], vv, t2 - (k - 1) * W)
        acc_ref[...] += lax.dot_general(
            h, ones, (((1,), (0,)), ((), ())),
            preferred_element_type=jnp.float32)

    @pl.when(jnp.logical_and(j >= 2, j < _GEND))
    def _middle():
        t2 = t_ref[...]
        vv = v_ref[...]
        h = None
        for k in range(K):
            tl = t2 - (3 + K * (j - 2) + k) * W
            hk = _hit(x_refs[k][...], vv, tl)
            h = hk if h is None else h + hk
        acc_ref[...] += lax.dot_general(
            h, ones, (((1,), (0,)), ((), ())),
            preferred_element_type=jnp.float32)

    @pl.when(j == _GEND)
    def _last():
        t2 = t_ref[...]
        vv = v_ref[...]
        h = None
        for k in range(_LAST_VALID):
            tl = t2 - (3 + K * (j - 2) + k) * W
            hk = _hit(x_refs[k][...], vv, tl)
            h = hk if h is None else h + hk
        rank = acc_ref[...] + lax.dot_general(
            h, ones, (((1,), (0,)), ((), ())),
            preferred_element_type=jnp.float32)
        out1_ref[0, 0] = jnp.sum(jnp.where(rank < 1.0, 1.0, 0.0)) * (100.0 / B)
        out5_ref[0, 0] = jnp.sum(jnp.where(rank < 5.0, 1.0, 0.0)) * (100.0 / B)


def _mk_imap(k):
    first = NB - 1 if k == 0 else k - 1

    def imap(j, ts):
        b = jnp.minimum(3 + K * (j - 2) + k, NB - 2)
        return (0, jnp.where(j <= 1, first, b))

    return imap


def _topk_acc(x, t2):
    grid_spec = pltpu.PrefetchScalarGridSpec(
        num_scalar_prefetch=1,
        grid=(_GRID,),
        in_specs=(
            [pl.BlockSpec((B, W), _mk_imap(k)) for k in range(K)]
            + [pl.BlockSpec(memory_space=pl.ANY),
               pl.BlockSpec((B, 1), lambda j, ts: (0, 0))]
        ),
        out_specs=[
            pl.BlockSpec(memory_space=pltpu.SMEM),
            pl.BlockSpec(memory_space=pltpu.SMEM),
        ],
        scratch_shapes=[
            pltpu.VMEM((B, 1), jnp.float32),
            pltpu.VMEM((B, 1), jnp.float32),
            pltpu.VMEM((B, 8, 128), jnp.float32),
            pltpu.SemaphoreType.DMA,
        ],
    )
    return pl.pallas_call(
        _topk_kernel,
        grid_spec=grid_spec,
        out_shape=[
            jax.ShapeDtypeStruct((1, 1), jnp.float32),
            jax.ShapeDtypeStruct((1, 1), jnp.float32),
        ],
        compiler_params=pltpu.CompilerParams(
            dimension_semantics=("arbitrary",)),
    )(t2.reshape(B), *([x] * K), x, t2)


def kernel(output, target):
    t32 = target.astype(jnp.int32)
    r1, r5 = _topk_acc(output, t32.reshape(B, 1))
    return (r1.reshape(1), r5.reshape(1))
